# Initial kernel scaffold; baseline (speedup 1.0000x reference)
import jax
import jax.numpy as jnp
from jax.experimental import pallas as pl


def kernel(x, edge_index0, edge_index1, W_self1, W_neigh1, b1, W_self2, W_neigh2, b2):
    raise NotImplementedError("write your pallas kernel here")


# speedup vs baseline: 8.0715x; 8.0715x over previous
import functools

import jax
import jax.numpy as jnp
from jax import lax
from jax.experimental import pallas as pl
from jax.experimental.pallas import tpu as pltpu
from jax.experimental.pallas import tpu_sc as plsc

N = 10000
E = 320000
D = 128

NC = 2
NS = 16
NW = NC * NS
CHUNK = 128
CPW = 80
EPAD = NW * CPW * CHUNK
NPAD = 10112
NDEG = 10240


def _sc_agg_body(h_hbm, src_hbm, dst_hbm, z2_hbm, z1_hbm,
                 agg0_out, agg1_out, deg0_out, deg1_out,
                 srcall, dstall, rows, ones_v, sem, agg_sh, deg_sh):
    c = lax.axis_index("c")
    s = lax.axis_index("s")
    wid = c * NS + s
    cstart = wid * CPW

    pltpu.sync_copy(src_hbm.at[pl.ds(cstart, CPW)], srcall)
    pltpu.sync_copy(dst_hbm.at[pl.ds(cstart, CPW)], dstall)

    rows_sl = pl.ds(s * (NPAD // NS), NPAD // NS)
    deg_sl = pl.ds(s * (NDEG // NS), NDEG // NS)
    pltpu.sync_copy(z2_hbm.at[rows_sl], agg_sh.at[rows_sl])
    pltpu.sync_copy(z1_hbm.at[deg_sl], deg_sh.at[deg_sl])

    for k in range(CHUNK // 16):
        ones_v[pl.ds(k * 16, 16)] = jnp.ones((16,), jnp.float32)

    plsc.subcore_barrier()

    def body(j, carry):
        pltpu.async_copy(h_hbm.at[srcall.at[j]], rows, sem).wait()
        pltpu.sync_copy(rows, agg_sh.at[dstall.at[j]], add=True)
        pltpu.sync_copy(ones_v, deg_sh.at[dstall.at[j]], add=True)
        return carry

    lax.fori_loop(0, CPW, body, 0)

    plsc.subcore_barrier()

    @pl.when(c == 0)
    def _():
        pltpu.sync_copy(agg_sh.at[rows_sl], agg0_out.at[rows_sl])
        pltpu.sync_copy(deg_sh.at[deg_sl], deg0_out.at[deg_sl])

    @pl.when(c == 1)
    def _():
        pltpu.sync_copy(agg_sh.at[rows_sl], agg1_out.at[rows_sl])
        pltpu.sync_copy(deg_sh.at[deg_sl], deg1_out.at[deg_sl])


_sc_agg = pl.kernel(
    _sc_agg_body,
    out_type=[
        jax.ShapeDtypeStruct((NPAD, D), jnp.float32),
        jax.ShapeDtypeStruct((NPAD, D), jnp.float32),
        jax.ShapeDtypeStruct((NDEG,), jnp.float32),
        jax.ShapeDtypeStruct((NDEG,), jnp.float32),
    ],
    mesh=plsc.VectorSubcoreMesh(core_axis_name="c", subcore_axis_name="s"),
    scratch_types=[
        pltpu.VMEM((CPW, CHUNK), jnp.int32),
        pltpu.VMEM((CPW, CHUNK), jnp.int32),
        pltpu.VMEM((CHUNK, D), jnp.float32),
        pltpu.VMEM((CHUNK,), jnp.float32),
        pltpu.SemaphoreType.DMA,
        pltpu.VMEM_SHARED((NPAD, D), jnp.float32),
        pltpu.VMEM_SHARED((NDEG,), jnp.float32),
    ],
)


def _tc_layer_body(relu, x_ref, a0_ref, a1_ref, dg_ref, ws_ref, wn_ref, b_ref, o_ref):
    agg = a0_ref[...] + a1_ref[...]
    dg = dg_ref[...]
    dsum = jnp.maximum(dg[:, 0:1] + dg[:, 1:2], 1.0)
    mean = agg / dsum
    acc = jnp.dot(x_ref[...], ws_ref[...], preferred_element_type=jnp.float32)
    acc = acc + jnp.dot(mean, wn_ref[...], preferred_element_type=jnp.float32)
    acc = acc + b_ref[...]
    if relu:
        acc = jnp.maximum(acc, 0.0)
    o_ref[...] = acc


def _tc_layer(x, a0, a1, degT, Ws, Wn, b, relu):
    blk = 2000
    grid = (N // blk,)
    return pl.pallas_call(
        functools.partial(_tc_layer_body, relu),
        grid=grid,
        in_specs=[
            pl.BlockSpec((blk, D), lambda i: (i, 0)),
            pl.BlockSpec((blk, D), lambda i: (i, 0)),
            pl.BlockSpec((blk, D), lambda i: (i, 0)),
            pl.BlockSpec((blk, 2), lambda i: (i, 0)),
            pl.BlockSpec((D, D), lambda i: (0, 0)),
            pl.BlockSpec((D, D), lambda i: (0, 0)),
            pl.BlockSpec((1, D), lambda i: (0, 0)),
        ],
        out_specs=pl.BlockSpec((blk, D), lambda i: (i, 0)),
        out_shape=jax.ShapeDtypeStruct((N, D), jnp.float32),
    )(x, a0, a1, degT, Ws, Wn, b.reshape(1, D))


def _pad_edges(ei):
    src = ei[0]
    dst = ei[1]
    p = EPAD - E
    padi = jnp.arange(p, dtype=jnp.int32)
    src_p = jnp.concatenate([src, padi % 16])
    dst_p = jnp.concatenate([dst, N + padi % (NPAD - N)])
    return src_p.reshape(EPAD // CHUNK, CHUNK), dst_p.reshape(EPAD // CHUNK, CHUNK)


def kernel(x, edge_index0, edge_index1, W_self1, W_neigh1, b1, W_self2, W_neigh2, b2):
    z2 = jnp.zeros((NPAD, D), jnp.float32)
    z1 = jnp.zeros((NDEG,), jnp.float32)
    s0, d0 = _pad_edges(edge_index0)
    s1, d1 = _pad_edges(edge_index1)

    a0, a1, g0, g1 = _sc_agg(x, s0, d0, z2, z1)
    degT1 = jnp.stack([g0[:N], g1[:N]], axis=1)
    h = _tc_layer(x, a0, a1, degT1, W_self1, W_neigh1, b1, True)

    a0b, a1b, g0b, g1b = _sc_agg(h, s1, d1, z2, z1)
    degT2 = jnp.stack([g0b[:N], g1b[:N]], axis=1)
    out = _tc_layer(h, a0b, a1b, degT2, W_self2, W_neigh2, b2, False)
    return out


# speedup vs baseline: 10.2611x; 1.2713x over previous
import functools

import jax
import jax.numpy as jnp
from jax import lax
from jax.experimental import pallas as pl
from jax.experimental.pallas import tpu as pltpu
from jax.experimental.pallas import tpu_sc as plsc

N = 10000
E = 320000
D = 128

NC = 2
NS = 16
NW = NC * NS
CHUNK = 128
CPW = 80
EPAD = NW * CPW * CHUNK
NPAD = 10112
NDEG = 10240


def _sc_agg_body(h_hbm, src_hbm, dst_hbm, z2_hbm, z1_hbm,
                 agg0_out, agg1_out, deg0_out, deg1_out,
                 srcall, dstall, rows_a, rows_b, ones_v, sem_a, sem_b,
                 agg_sh, deg_sh):
    c = lax.axis_index("c")
    s = lax.axis_index("s")
    wid = c * NS + s
    cstart = wid * CPW

    rows_sl = pl.ds(s * (NPAD // NS), NPAD // NS)
    deg_sl = pl.ds(s * (NDEG // NS), NDEG // NS)
    pltpu.sync_copy(z2_hbm.at[rows_sl], agg_sh.at[rows_sl])
    pltpu.sync_copy(z1_hbm.at[deg_sl], deg_sh.at[deg_sl])

    for k in range(CHUNK // 16):
        ones_v[pl.ds(k * 16, 16)] = jnp.ones((16,), jnp.float32)

    plsc.subcore_barrier()

    half_c = CPW // 2
    for half in range(2):
        hstart = cstart + half * half_c
        pltpu.sync_copy(src_hbm.at[pl.ds(hstart, half_c)], srcall)
        pltpu.sync_copy(dst_hbm.at[pl.ds(hstart, half_c)], dstall)
        pltpu.async_copy(h_hbm.at[srcall.at[0]], rows_a, sem_a)

        def body(k, carry):
            j0 = 2 * k
            j1 = j0 + 1
            pltpu.make_async_copy(h_hbm.at[srcall.at[j0]], rows_a, sem_a).wait()
            pltpu.async_copy(h_hbm.at[srcall.at[j1]], rows_b, sem_b)
            pltpu.sync_copy(rows_a, agg_sh.at[dstall.at[j0]], add=True)
            pltpu.sync_copy(ones_v, deg_sh.at[dstall.at[j0]], add=True)
            pltpu.make_async_copy(h_hbm.at[srcall.at[j1]], rows_b, sem_b).wait()

            @pl.when(k < half_c // 2 - 1)
            def _():
                pltpu.async_copy(h_hbm.at[srcall.at[j0 + 2]], rows_a, sem_a)

            pltpu.sync_copy(rows_b, agg_sh.at[dstall.at[j1]], add=True)
            pltpu.sync_copy(ones_v, deg_sh.at[dstall.at[j1]], add=True)
            return carry

        lax.fori_loop(0, half_c // 2, body, 0)

    plsc.subcore_barrier()

    @pl.when(c == 0)
    def _():
        pltpu.sync_copy(agg_sh.at[rows_sl], agg0_out.at[rows_sl])
        pltpu.sync_copy(deg_sh.at[deg_sl], deg0_out.at[deg_sl])

    @pl.when(c == 1)
    def _():
        pltpu.sync_copy(agg_sh.at[rows_sl], agg1_out.at[rows_sl])
        pltpu.sync_copy(deg_sh.at[deg_sl], deg1_out.at[deg_sl])


_sc_agg = pl.kernel(
    _sc_agg_body,
    out_type=[
        jax.ShapeDtypeStruct((NPAD, D), jnp.float32),
        jax.ShapeDtypeStruct((NPAD, D), jnp.float32),
        jax.ShapeDtypeStruct((NDEG,), jnp.float32),
        jax.ShapeDtypeStruct((NDEG,), jnp.float32),
    ],
    mesh=plsc.VectorSubcoreMesh(core_axis_name="c", subcore_axis_name="s"),
    scratch_types=[
        pltpu.VMEM((CPW // 2, CHUNK), jnp.int32),
        pltpu.VMEM((CPW // 2, CHUNK), jnp.int32),
        pltpu.VMEM((CHUNK, D), jnp.float32),
        pltpu.VMEM((CHUNK, D), jnp.float32),
        pltpu.VMEM((CHUNK,), jnp.float32),
        pltpu.SemaphoreType.DMA,
        pltpu.SemaphoreType.DMA,
        pltpu.VMEM_SHARED((NPAD, D), jnp.float32),
        pltpu.VMEM_SHARED((NDEG,), jnp.float32),
    ],
)


def _tc_layer_body(relu, x_ref, a0_ref, a1_ref, g0_ref, g1_ref, ws_ref, wn_ref, b_ref, o_ref):
    agg = a0_ref[...] + a1_ref[...]
    dsum = jnp.maximum(g0_ref[...] + g1_ref[...], 1.0)
    mean = agg / dsum
    acc = jnp.dot(x_ref[...], ws_ref[...], preferred_element_type=jnp.float32)
    acc = acc + jnp.dot(mean, wn_ref[...], preferred_element_type=jnp.float32)
    acc = acc + b_ref[...]
    if relu:
        acc = jnp.maximum(acc, 0.0)
    o_ref[...] = acc


def _tc_layer(x, a0, a1, g0, g1, Ws, Wn, b, relu):
    blk = 2000
    grid = (N // blk,)
    return pl.pallas_call(
        functools.partial(_tc_layer_body, relu),
        grid=grid,
        in_specs=[
            pl.BlockSpec((blk, D), lambda i: (i, 0)),
            pl.BlockSpec((blk, D), lambda i: (i, 0)),
            pl.BlockSpec((blk, D), lambda i: (i, 0)),
            pl.BlockSpec((blk, 1), lambda i: (i, 0)),
            pl.BlockSpec((blk, 1), lambda i: (i, 0)),
            pl.BlockSpec((D, D), lambda i: (0, 0)),
            pl.BlockSpec((D, D), lambda i: (0, 0)),
            pl.BlockSpec((1, D), lambda i: (0, 0)),
        ],
        out_specs=pl.BlockSpec((blk, D), lambda i: (i, 0)),
        out_shape=jax.ShapeDtypeStruct((N, D), jnp.float32),
    )(x, a0, a1, g0.reshape(NDEG, 1), g1.reshape(NDEG, 1), Ws, Wn, b.reshape(1, D))


def _pad_edges(ei):
    src = ei[0]
    dst = ei[1]
    p = EPAD - E
    padi = jnp.arange(p, dtype=jnp.int32)
    src_p = jnp.concatenate([src, padi % 16])
    dst_p = jnp.concatenate([dst, N + padi % (NPAD - N)])
    return src_p.reshape(EPAD // CHUNK, CHUNK), dst_p.reshape(EPAD // CHUNK, CHUNK)


def kernel(x, edge_index0, edge_index1, W_self1, W_neigh1, b1, W_self2, W_neigh2, b2):
    z2 = jnp.zeros((NPAD, D), jnp.float32)
    z1 = jnp.zeros((NDEG,), jnp.float32)
    s0, d0 = _pad_edges(edge_index0)
    s1, d1 = _pad_edges(edge_index1)

    a0, a1, g0, g1 = _sc_agg(x, s0, d0, z2, z1)
    h = _tc_layer(x, a0, a1, g0, g1, W_self1, W_neigh1, b1, True)

    a0b, a1b, g0b, g1b = _sc_agg(h, s1, d1, z2, z1)
    out = _tc_layer(h, a0b, a1b, g0b, g1b, W_self2, W_neigh2, b2, False)
    return out


# speedup vs baseline: 11.6875x; 1.1390x over previous
import functools

import jax
import jax.numpy as jnp
from jax import lax
from jax.experimental import pallas as pl
from jax.experimental.pallas import tpu as pltpu
from jax.experimental.pallas import tpu_sc as plsc

N = 10000
E = 320000
D = 128

NC = 2
NS = 16
NW = NC * NS
CW = 64
CPW = 160
EPAD = NW * CPW * CW
NPAD = 10112
NDEG = 10240
NBUF = 4


def _sc_agg_body(h_hbm, src_hbm, dst_hbm, z2_hbm, z1_hbm,
                 agg0_out, agg1_out, deg0_out, deg1_out,
                 srcall, dstall, b0, b1, b2, b3,
                 g0, g1, g2, g3, s0, s1, s2, s3,
                 ones_v, agg_sh, deg_sh):
    c = lax.axis_index("c")
    s = lax.axis_index("s")
    wid = c * NS + s
    cstart = wid * CPW

    bufs = (b0, b1, b2, b3)
    gsems = (g0, g1, g2, g3)
    ssems = (s0, s1, s2, s3)

    rows_sl = pl.ds(s * (NPAD // NS), NPAD // NS)
    deg_sl = pl.ds(s * (NDEG // NS), NDEG // NS)
    pltpu.sync_copy(z2_hbm.at[rows_sl], agg_sh.at[rows_sl])
    pltpu.sync_copy(z1_hbm.at[deg_sl], deg_sh.at[deg_sl])

    for k in range(CW // 16):
        ones_v[pl.ds(k * 16, 16)] = jnp.ones((16,), jnp.float32)

    plsc.subcore_barrier()

    half_c = CPW // 4
    for half in range(4):
        hstart = cstart + half * half_c
        pltpu.sync_copy(src_hbm.at[pl.ds(hstart, half_c)], srcall)
        pltpu.sync_copy(dst_hbm.at[pl.ds(hstart, half_c)], dstall)

        def body(k, carry):
            for i in range(NBUF):
                j = NBUF * k + i
                i2 = (i + 2) % NBUF

                @pl.when(j >= NBUF)
                def _():
                    pltpu.make_async_copy(
                        bufs[i], agg_sh.at[dstall.at[0]], ssems[i]).wait()

                @pl.when(j < half_c)
                def _():
                    pltpu.async_copy(h_hbm.at[srcall.at[j]], bufs[i], gsems[i])

                jm2 = j - 2

                @pl.when((j >= 2) & (j < half_c + 2))
                def _():
                    pltpu.make_async_copy(
                        h_hbm.at[srcall.at[jm2]], bufs[i2], gsems[i2]).wait()
                    pltpu.async_copy(
                        bufs[i2], agg_sh.at[dstall.at[jm2]], ssems[i2], add=True)
                    pltpu.sync_copy(ones_v, deg_sh.at[dstall.at[jm2]], add=True)
            return carry

        lax.fori_loop(0, half_c // NBUF + 1, body, 0)

    plsc.subcore_barrier()

    @pl.when(c == 0)
    def _():
        pltpu.sync_copy(agg_sh.at[rows_sl], agg0_out.at[rows_sl])
        pltpu.sync_copy(deg_sh.at[deg_sl], deg0_out.at[deg_sl])

    @pl.when(c == 1)
    def _():
        pltpu.sync_copy(agg_sh.at[rows_sl], agg1_out.at[rows_sl])
        pltpu.sync_copy(deg_sh.at[deg_sl], deg1_out.at[deg_sl])


_sc_agg = pl.kernel(
    _sc_agg_body,
    out_type=[
        jax.ShapeDtypeStruct((NPAD, D), jnp.float32),
        jax.ShapeDtypeStruct((NPAD, D), jnp.float32),
        jax.ShapeDtypeStruct((NDEG,), jnp.float32),
        jax.ShapeDtypeStruct((NDEG,), jnp.float32),
    ],
    mesh=plsc.VectorSubcoreMesh(core_axis_name="c", subcore_axis_name="s"),
    scratch_types=[
        pltpu.VMEM((CPW // 4, CW), jnp.int32),
        pltpu.VMEM((CPW // 4, CW), jnp.int32),
        pltpu.VMEM((CW, D), jnp.float32),
        pltpu.VMEM((CW, D), jnp.float32),
        pltpu.VMEM((CW, D), jnp.float32),
        pltpu.VMEM((CW, D), jnp.float32),
        pltpu.SemaphoreType.DMA,
        pltpu.SemaphoreType.DMA,
        pltpu.SemaphoreType.DMA,
        pltpu.SemaphoreType.DMA,
        pltpu.SemaphoreType.DMA,
        pltpu.SemaphoreType.DMA,
        pltpu.SemaphoreType.DMA,
        pltpu.SemaphoreType.DMA,
        pltpu.VMEM((CW,), jnp.float32),
        pltpu.VMEM_SHARED((NPAD, D), jnp.float32),
        pltpu.VMEM_SHARED((NDEG,), jnp.float32),
    ],
)


def _tc_layer_body(relu, x_ref, a0_ref, a1_ref, g0_ref, g1_ref, ws_ref, wn_ref, b_ref, o_ref):
    agg = a0_ref[...] + a1_ref[...]
    dsum = jnp.maximum(g0_ref[...] + g1_ref[...], 1.0)
    mean = agg / dsum
    acc = jnp.dot(x_ref[...], ws_ref[...], preferred_element_type=jnp.float32)
    acc = acc + jnp.dot(mean, wn_ref[...], preferred_element_type=jnp.float32)
    acc = acc + b_ref[...]
    if relu:
        acc = jnp.maximum(acc, 0.0)
    o_ref[...] = acc


def _tc_layer(x, a0, a1, g0, g1, Ws, Wn, b, relu):
    blk = 2000
    grid = (N // blk,)
    return pl.pallas_call(
        functools.partial(_tc_layer_body, relu),
        grid=grid,
        in_specs=[
            pl.BlockSpec((blk, D), lambda i: (i, 0)),
            pl.BlockSpec((blk, D), lambda i: (i, 0)),
            pl.BlockSpec((blk, D), lambda i: (i, 0)),
            pl.BlockSpec((blk, 1), lambda i: (i, 0)),
            pl.BlockSpec((blk, 1), lambda i: (i, 0)),
            pl.BlockSpec((D, D), lambda i: (0, 0)),
            pl.BlockSpec((D, D), lambda i: (0, 0)),
            pl.BlockSpec((1, D), lambda i: (0, 0)),
        ],
        out_specs=pl.BlockSpec((blk, D), lambda i: (i, 0)),
        out_shape=jax.ShapeDtypeStruct((N, D), jnp.float32),
    )(x, a0, a1, g0.reshape(NDEG, 1), g1.reshape(NDEG, 1), Ws, Wn, b.reshape(1, D))


def _pad_edges(ei):
    src = ei[0]
    dst = ei[1]
    p = EPAD - E
    padi = jnp.arange(p, dtype=jnp.int32)
    src_p = jnp.concatenate([src, padi % 16])
    dst_p = jnp.concatenate([dst, N + padi % (NPAD - N)])
    return src_p.reshape(EPAD // CW, CW), dst_p.reshape(EPAD // CW, CW)


def kernel(x, edge_index0, edge_index1, W_self1, W_neigh1, b1, W_self2, W_neigh2, b2):
    z2 = jnp.zeros((NPAD, D), jnp.float32)
    z1 = jnp.zeros((NDEG,), jnp.float32)
    s0, d0 = _pad_edges(edge_index0)
    s1, d1 = _pad_edges(edge_index1)

    a0, a1, g0, g1 = _sc_agg(x, s0, d0, z2, z1)
    h = _tc_layer(x, a0, a1, g0, g1, W_self1, W_neigh1, b1, True)

    a0b, a1b, g0b, g1b = _sc_agg(h, s1, d1, z2, z1)
    out = _tc_layer(h, a0b, a1b, g0b, g1b, W_self2, W_neigh2, b2, False)
    return out
